# baseline (speedup 1.0000x reference)
import functools

import jax
import jax.numpy as jnp
from jax import lax
from jax.experimental import pallas as pl
from jax.experimental.pallas import tpu as pltpu
from jax.experimental.pallas import tpu_sc as plsc

OFF_THRESH = 0.5
INF = 1e10

H = 384
W = 384
B = 2
HP = H + 2
WBUF = 416

R_TC = 192
RPW = (H - R_TC) // 16
S1_ROWS = RPW + 2
Z_ROWS = S1_ROWS + 2

C28 = 1 << 28
E_CASE_B = -(4 << 28) + C28 - 16 + 4
E_CASE_CD = 1 << 29


def _sc_body(xp_hbm, c_hbm, out_hbm,
             zbuf, xbuf, ybuf, mbuf, ebuf, ox, oy, oz, om, cbuf, dsem):
    wi = lax.axis_index("s")
    b = lax.axis_index("c")
    r0 = R_TC + RPW * wi
    ch = b * 4

    copies = [
        pltpu.async_copy(c_hbm, cbuf, dsem),
        pltpu.async_copy(
            xp_hbm.at[pl.ds((ch + 0) * 388 + r0 + 1, S1_ROWS), :], xbuf, dsem),
        pltpu.async_copy(
            xp_hbm.at[pl.ds((ch + 1) * 388 + r0 + 1, S1_ROWS), :], ybuf, dsem),
        pltpu.async_copy(
            xp_hbm.at[pl.ds((ch + 2) * 388 + r0, Z_ROWS), :], zbuf, dsem),
        pltpu.async_copy(
            xp_hbm.at[pl.ds((ch + 3) * 388 + r0 + 1, S1_ROWS), :], mbuf, dsem),
    ]
    for c in copies:
        c.wait()

    a = [cbuf[i, :] for i in range(9)]
    iota_i = lax.iota(jnp.int32, 16)
    iota = iota_i.astype(jnp.float32)

    def s1_row(rr, _):
        vf = (r0 + rr - 1).astype(jnp.float32)
        rowx = a[1] * vf + a[2]
        rowy = a[4] * vf + a[5]
        rowz = a[7] * vf + a[8]

        @plsc.parallel_loop(0, 25, unroll=2)
        def s1_chunk(j):
            c0 = 16 * j
            bc = c0 + 1
            u = iota + (c0 - 1).astype(jnp.float32)
            rx_c = a[0] * u + rowx
            ry_c = a[3] * u + rowy
            rz_c = a[6] * u + rowz
            xc = xbuf[rr, pl.ds(bc, 16)]
            yc = ybuf[rr, pl.ds(bc, 16)]
            zc = zbuf[rr + 1, pl.ds(bc, 16)]
            mc = mbuf[rr, pl.ds(bc, 16)]
            ds = []
            for k in range(9):
                dr, dc = k // 3 - 1, k % 3 - 1
                zn = zbuf[rr + 1 + dr, pl.ds(bc + dc, 16)]
                rx = rx_c + cbuf[9 + k, :]
                ry = ry_c + cbuf[18 + k, :]
                rz = rz_c + cbuf[27 + k, :]
                dx = rx * zn - xc
                dy = ry * zn - yc
                dz = rz * zn - zc
                ds.append(dx * dx + dy * dy + dz * dz)
            pairs = [(ds[k], jnp.full((16,), k, jnp.int32)) for k in range(9)]
            while len(pairs) > 1:
                nxt = []
                for i in range(0, len(pairs) - 1, 2):
                    (da, ka), (db, kb) = pairs[i], pairs[i + 1]
                    m = db < da
                    nxt.append((jnp.where(m, db, da), jnp.where(m, kb, ka)))
                if len(pairs) % 2:
                    nxt.append(pairs[-1])
                pairs = nxt
            best_k = pairs[0][1]
            rmask = mc > OFF_THRESH
            zmask = zc > 0.0
            zb = (lax.bitcast_convert_type(zc, jnp.int32) >> 2) & ~15
            e_a = (best_k - 8) * jnp.int32(C28) + zb
            e = jnp.where(rmask,
                          jnp.where(zmask, e_a, jnp.int32(E_CASE_B)),
                          jnp.int32(E_CASE_CD))
            ebuf[rr, pl.ds(bc, 16)] = e

        return 0

    lax.fori_loop(0, S1_ROWS, s1_row, 0)

    def s2_row(rr2, _):
        xr = rr2 + 1

        @plsc.parallel_loop(0, W // 16, unroll=2)
        def s2_chunk(j2):
            b0 = 16 * j2 + 2
            e_c = ebuf[xr, pl.ds(b0, 16)]
            sent = jnp.full((16,), C28, jnp.int32)
            keys = []
            for k in range(9):
                ro = -(k // 3 - 1)
                co = -(k % 3 - 1)
                if ro == 0 and co == 0:
                    e_q = e_c
                else:
                    e_q = ebuf[xr + ro, pl.ds(b0 + co, 16)]
                cand = e_q - jnp.int32((k - 8) * C28)
                valid = (cand > 0) & (cand < sent)
                keys.append(jnp.where(valid, cand | k if k else cand, sent))
            while len(keys) > 1:
                nxt = [jnp.minimum(keys[i], keys[i + 1])
                       for i in range(0, len(keys) - 1, 2)]
                if len(keys) % 2:
                    nxt.append(keys[-1])
                keys = nxt
            best = keys[0]
            found = best < sent
            bk = jnp.where(found, best & 15, 4)
            s = (bk * 11) >> 5
            t = bk - 3 * s
            rowq = (xr + 1) - s
            colq = iota_i + (b0 + 1) - t
            xq = plsc.load_gather(xbuf, [rowq, colq])
            yq = plsc.load_gather(ybuf, [rowq, colq])
            zq = plsc.load_gather(zbuf, [rowq + 1, colq])
            mq = plsc.load_gather(mbuf, [rowq, colq])
            t4 = e_c + jnp.int32(4 * C28)
            c4 = (t4 > 0) & (t4 < C28)
            cm4 = c4 | (e_c == jnp.int32(E_CASE_CD))
            fx = found | c4
            fm = found | cm4
            zero = jnp.zeros((16,), jnp.float32)
            ox[rr2, pl.ds(16 * j2, 16)] = jnp.where(fx, xq, zero)
            oy[rr2, pl.ds(16 * j2, 16)] = jnp.where(fx, yq, zero)
            oz[rr2, pl.ds(16 * j2, 16)] = jnp.where(fx, zq, zero)
            om[rr2, pl.ds(16 * j2, 16)] = jnp.where(fm, mq, zero)

        return 0

    lax.fori_loop(0, RPW, s2_row, 0)

    base = b * 4 * H + r0
    ocopies = [
        pltpu.async_copy(ox, out_hbm.at[pl.ds(base + 0 * H, RPW), :], dsem),
        pltpu.async_copy(oy, out_hbm.at[pl.ds(base + 1 * H, RPW), :], dsem),
        pltpu.async_copy(oz, out_hbm.at[pl.ds(base + 2 * H, RPW), :], dsem),
        pltpu.async_copy(om, out_hbm.at[pl.ds(base + 3 * H, RPW), :], dsem),
    ]
    for c in ocopies:
        c.wait()


_smap_sc = functools.partial(
    pl.kernel,
    out_type=jax.ShapeDtypeStruct((B * 4 * H, W), jnp.float32),
    mesh=plsc.VectorSubcoreMesh(core_axis_name="c", subcore_axis_name="s"),
    compiler_params=pltpu.CompilerParams(
        use_tc_tiling_on_sc=False, needs_layout_passes=False),
    name="smap_sc",
    scratch_types=[
        pltpu.VMEM((Z_ROWS, WBUF), jnp.float32),
        pltpu.VMEM((S1_ROWS, WBUF), jnp.float32),
        pltpu.VMEM((S1_ROWS, WBUF), jnp.float32),
        pltpu.VMEM((S1_ROWS, WBUF), jnp.float32),
        pltpu.VMEM((S1_ROWS, WBUF), jnp.int32),
        pltpu.VMEM((RPW, W), jnp.float32),
        pltpu.VMEM((RPW, W), jnp.float32),
        pltpu.VMEM((RPW, W), jnp.float32),
        pltpu.VMEM((RPW, W), jnp.float32),
        pltpu.VMEM((36, 16), jnp.float32),
        pltpu.SemaphoreType.DMA,
    ],
)(_sc_body)


def _tc_body(xp_ref, c_ref, out_ref):
    s1r = R_TC + 2
    a = [c_ref[i] for i in range(9)]
    sx = [c_ref[9 + k] for k in range(9)]
    sy = [c_ref[18 + k] for k in range(9)]
    sz = [c_ref[27 + k] for k in range(9)]
    u = lax.broadcasted_iota(jnp.int32, (s1r, 400), 1).astype(jnp.float32) - 1.0
    v = lax.broadcasted_iota(jnp.int32, (s1r, 400), 0).astype(jnp.float32) - 1.0
    rx_c = a[0] * u + (a[1] * v + a[2])
    ry_c = a[3] * u + (a[4] * v + a[5])
    rz_c = a[6] * u + (a[7] * v + a[8])
    for b in range(B):
        X = xp_ref[b, 0]
        Y = xp_ref[b, 1]
        Z = xp_ref[b, 2]
        M = xp_ref[b, 3]
        xc = X[1:1 + s1r, 1:401]
        yc = Y[1:1 + s1r, 1:401]
        zc = Z[1:1 + s1r, 1:401]
        mc = M[1:1 + s1r, 1:401]
        pairs = []
        for k in range(9):
            dr, dc = k // 3 - 1, k % 3 - 1
            zn = Z[1 + dr:1 + dr + s1r, 1 + dc:401 + dc]
            rx = rx_c + sx[k]
            ry = ry_c + sy[k]
            rz = rz_c + sz[k]
            dx = rx * zn - xc
            dy = ry * zn - yc
            dz = rz * zn - zc
            d = dx * dx + dy * dy + dz * dz
            pairs.append((d, jnp.full(d.shape, k, jnp.int32)))
        while len(pairs) > 1:
            nxt = []
            for i in range(0, len(pairs) - 1, 2):
                (da, ka), (db, kb) = pairs[i], pairs[i + 1]
                m = db < da
                nxt.append((jnp.where(m, db, da), jnp.where(m, kb, ka)))
            if len(pairs) % 2:
                nxt.append(pairs[-1])
            pairs = nxt
        best_k = pairs[0][1]
        rmask = mc > OFF_THRESH
        zmask = zc > 0.0
        ixz = jnp.where(rmask, jnp.where(zmask, best_k, 4), 9)
        irm = jnp.where(rmask & zmask, best_k, 4)

        def s1s(P, s, t):
            return P[2 - s:2 - s + R_TC, 2 - t:2 - t + W]

        def pls(P, s, t):
            return P[3 - s:3 - s + R_TC, 3 - t:3 - t + W]

        ixz_c = ixz[1:1 + R_TC, 1:1 + W]
        irm_c = irm[1:1 + R_TC, 1:1 + W]
        inf2 = jnp.full((R_TC, W), INF, jnp.float32)
        zero2 = jnp.zeros((R_TC, W), jnp.float32)
        c4 = ixz_c == 4
        bx = jnp.where(c4, pls(X, 1, 1), zero2)
        by = jnp.where(c4, pls(Y, 1, 1), zero2)
        bz = jnp.where(c4, pls(Z, 1, 1), zero2)
        brm = jnp.where(irm_c == 4, pls(M, 1, 1), zero2)
        best = inf2
        for k in range(9):
            s, t = k // 3, k % 3
            ixz_q = s1s(ixz, s, t)
            zq = pls(Z, s, t)
            cand = jnp.where(ixz_q == k, zq, zero2)
            cand = jnp.where(cand > 0.0, cand, inf2)
            m = cand < best
            best = jnp.where(m, cand, best)
            bx = jnp.where(m, pls(X, s, t), bx)
            by = jnp.where(m, pls(Y, s, t), by)
            bz = jnp.where(m, zq, bz)
            brm = jnp.where(m, pls(M, s, t), brm)
        out_ref[b, 0] = bx
        out_ref[b, 1] = by
        out_ref[b, 2] = bz
        out_ref[b, 3] = brm


_smap_tc = pl.pallas_call(
    _tc_body,
    out_shape=jax.ShapeDtypeStruct((B, 4, R_TC, W), jnp.float32),
    in_specs=[
        pl.BlockSpec(memory_space=pltpu.VMEM),
        pl.BlockSpec(memory_space=pltpu.SMEM),
    ],
    out_specs=pl.BlockSpec(memory_space=pltpu.VMEM),
    name="smap_tc",
)


def kernel(x, camera_matrix):
    k_inv = jnp.linalg.inv(camera_matrix)
    dc = jnp.array([k % 3 - 1 for k in range(9)], jnp.float32)
    dr = jnp.array([k // 3 - 1 for k in range(9)], jnp.float32)
    sx = k_inv[0, 0] * dc + k_inv[0, 1] * dr
    sy = k_inv[1, 0] * dc + k_inv[1, 1] * dr
    sz = k_inv[2, 0] * dc + k_inv[2, 1] * dr
    consts = jnp.concatenate([k_inv.reshape(9), sx, sy, sz])
    xp = jnp.pad(x, ((0, 0), (0, 0), (2, 2), (2, WBUF - W - 2)))
    xp2 = xp.reshape(B * 4 * (HP + 2), WBUF)
    consts16 = jnp.repeat(consts.reshape(36, 1), 16, axis=1)
    out_sc = _smap_sc(xp2, consts16)
    out_tc = _smap_tc(xp, consts)
    return lax.dynamic_update_slice(
        out_sc.reshape(B, 4, H, W), out_tc, (0, 0, 0, 0))


# speedup vs baseline: 1.0048x; 1.0048x over previous
import functools

import jax
import jax.numpy as jnp
from jax import lax
from jax.experimental import pallas as pl
from jax.experimental.pallas import tpu as pltpu
from jax.experimental.pallas import tpu_sc as plsc

OFF_THRESH = 0.5
INF = 1e10

H = 384
W = 384
B = 2
HP = H + 2
WBUF = 416

R_TC = 192
RPW = (H - R_TC) // 16
S1_ROWS = RPW + 2
Z_ROWS = S1_ROWS + 2

C28 = 1 << 28
E_CASE_B = -(4 << 28) + C28 - 1
E_CASE_CD = 1 << 29


def _sc_body(xp_hbm, c_hbm, out_hbm,
             zbuf, xbuf, ybuf, mbuf, ebuf, ox, oy, oz, om, cbuf, dsem):
    wi = lax.axis_index("s")
    b = lax.axis_index("c")
    r0 = R_TC + RPW * wi
    ch = b * 4

    copies = [
        pltpu.async_copy(c_hbm, cbuf, dsem),
        pltpu.async_copy(
            xp_hbm.at[pl.ds((ch + 0) * 388 + r0 + 1, S1_ROWS), :], xbuf, dsem),
        pltpu.async_copy(
            xp_hbm.at[pl.ds((ch + 1) * 388 + r0 + 1, S1_ROWS), :], ybuf, dsem),
        pltpu.async_copy(
            xp_hbm.at[pl.ds((ch + 2) * 388 + r0, Z_ROWS), :], zbuf, dsem),
        pltpu.async_copy(
            xp_hbm.at[pl.ds((ch + 3) * 388 + r0 + 1, S1_ROWS), :], mbuf, dsem),
    ]
    for c in copies:
        c.wait()

    a = [cbuf[i, :] for i in range(9)]
    iota_i = lax.iota(jnp.int32, 16)
    iota = iota_i.astype(jnp.float32)

    @plsc.parallel_loop(0, S1_ROWS)
    def s1_row(rr):
        vf = (r0 + rr - 1).astype(jnp.float32)
        rowx = a[1] * vf + a[2]
        rowy = a[4] * vf + a[5]
        rowz = a[7] * vf + a[8]

        @plsc.parallel_loop(0, 25, unroll=2)
        def s1_chunk(j):
            c0 = 16 * j
            bc = c0 + 1
            u = iota + (c0 - 1).astype(jnp.float32)
            rx_c = a[0] * u + rowx
            ry_c = a[3] * u + rowy
            rz_c = a[6] * u + rowz
            xc = xbuf[rr, pl.ds(bc, 16)]
            yc = ybuf[rr, pl.ds(bc, 16)]
            zc = zbuf[rr + 1, pl.ds(bc, 16)]
            mc = mbuf[rr, pl.ds(bc, 16)]
            ds = []
            for k in range(9):
                dr, dc = k // 3 - 1, k % 3 - 1
                zn = zbuf[rr + 1 + dr, pl.ds(bc + dc, 16)]
                rx = rx_c + cbuf[9 + k, :]
                ry = ry_c + cbuf[18 + k, :]
                rz = rz_c + cbuf[27 + k, :]
                dx = rx * zn - xc
                dy = ry * zn - yc
                dz = rz * zn - zc
                ds.append(dx * dx + dy * dy + dz * dz)
            pairs = [(ds[k], jnp.full((16,), k, jnp.int32)) for k in range(9)]
            while len(pairs) > 1:
                nxt = []
                for i in range(0, len(pairs) - 1, 2):
                    (da, ka), (db, kb) = pairs[i], pairs[i + 1]
                    m = db < da
                    nxt.append((jnp.where(m, db, da), jnp.where(m, kb, ka)))
                if len(pairs) % 2:
                    nxt.append(pairs[-1])
                pairs = nxt
            best_k = pairs[0][1]
            rmask = mc > OFF_THRESH
            zmask = zc > 0.0
            zb = lax.bitcast_convert_type(zc, jnp.int32) >> 2
            e_a = (best_k - 8) * jnp.int32(C28) + zb
            e = jnp.where(rmask,
                          jnp.where(zmask, e_a, jnp.int32(E_CASE_B)),
                          jnp.int32(E_CASE_CD))
            ebuf[rr, pl.ds(bc, 16)] = e

    @plsc.parallel_loop(0, RPW)
    def s2_row(rr2):
        xr = rr2 + 1

        @plsc.parallel_loop(0, W // 16, unroll=2)
        def s2_chunk(j2):
            b0 = 16 * j2 + 2
            e_c = ebuf[xr, pl.ds(b0, 16)]
            sent = jnp.full((16,), C28, jnp.int32)
            keys = []
            for k in range(9):
                ro = -(k // 3 - 1)
                co = -(k % 3 - 1)
                if ro == 0 and co == 0:
                    e_q = e_c
                else:
                    e_q = ebuf[xr + ro, pl.ds(b0 + co, 16)]
                cand = e_q - jnp.int32((k - 8) * C28)
                valid = (cand > 0) & (cand < sent)
                key = jnp.where(valid, cand, sent)
                keys.append((key, jnp.full((16,), k, jnp.int32)))
            while len(keys) > 1:
                nxt = []
                for i in range(0, len(keys) - 1, 2):
                    (da, ka), (db, kb) = keys[i], keys[i + 1]
                    m = db < da
                    nxt.append((jnp.where(m, db, da), jnp.where(m, kb, ka)))
                if len(keys) % 2:
                    nxt.append(keys[-1])
                keys = nxt
            best, bk = keys[0]
            found = best < sent
            bk = jnp.where(found, bk, 4)
            s = (bk * 11) >> 5
            t = bk - 3 * s
            rowq = (xr + 1) - s
            colq = iota_i + (b0 + 1) - t
            xq = plsc.load_gather(xbuf, [rowq, colq])
            yq = plsc.load_gather(ybuf, [rowq, colq])
            zq = plsc.load_gather(zbuf, [rowq + 1, colq])
            mq = plsc.load_gather(mbuf, [rowq, colq])
            t4 = e_c + jnp.int32(4 * C28)
            c4 = (t4 > 0) & (t4 < C28)
            cm4 = c4 | (e_c == jnp.int32(E_CASE_CD))
            fx = found | c4
            fm = found | cm4
            zero = jnp.zeros((16,), jnp.float32)
            ox[rr2, pl.ds(16 * j2, 16)] = jnp.where(fx, xq, zero)
            oy[rr2, pl.ds(16 * j2, 16)] = jnp.where(fx, yq, zero)
            oz[rr2, pl.ds(16 * j2, 16)] = jnp.where(fx, zq, zero)
            om[rr2, pl.ds(16 * j2, 16)] = jnp.where(fm, mq, zero)

    base = b * 4 * H + r0
    ocopies = [
        pltpu.async_copy(ox, out_hbm.at[pl.ds(base + 0 * H, RPW), :], dsem),
        pltpu.async_copy(oy, out_hbm.at[pl.ds(base + 1 * H, RPW), :], dsem),
        pltpu.async_copy(oz, out_hbm.at[pl.ds(base + 2 * H, RPW), :], dsem),
        pltpu.async_copy(om, out_hbm.at[pl.ds(base + 3 * H, RPW), :], dsem),
    ]
    for c in ocopies:
        c.wait()


_smap_sc = functools.partial(
    pl.kernel,
    out_type=jax.ShapeDtypeStruct((B * 4 * H, W), jnp.float32),
    mesh=plsc.VectorSubcoreMesh(core_axis_name="c", subcore_axis_name="s"),
    compiler_params=pltpu.CompilerParams(
        use_tc_tiling_on_sc=False, needs_layout_passes=False),
    name="smap_sc",
    scratch_types=[
        pltpu.VMEM((Z_ROWS, WBUF), jnp.float32),
        pltpu.VMEM((S1_ROWS, WBUF), jnp.float32),
        pltpu.VMEM((S1_ROWS, WBUF), jnp.float32),
        pltpu.VMEM((S1_ROWS, WBUF), jnp.float32),
        pltpu.VMEM((S1_ROWS, WBUF), jnp.int32),
        pltpu.VMEM((RPW, W), jnp.float32),
        pltpu.VMEM((RPW, W), jnp.float32),
        pltpu.VMEM((RPW, W), jnp.float32),
        pltpu.VMEM((RPW, W), jnp.float32),
        pltpu.VMEM((36, 16), jnp.float32),
        pltpu.SemaphoreType.DMA,
    ],
)(_sc_body)


def _tc_body(xp_ref, c_ref, out_ref):
    s1r = R_TC + 2
    a = [c_ref[i] for i in range(9)]
    sx = [c_ref[9 + k] for k in range(9)]
    sy = [c_ref[18 + k] for k in range(9)]
    sz = [c_ref[27 + k] for k in range(9)]
    u = lax.broadcasted_iota(jnp.int32, (s1r, 400), 1).astype(jnp.float32) - 1.0
    v = lax.broadcasted_iota(jnp.int32, (s1r, 400), 0).astype(jnp.float32) - 1.0
    rx_c = a[0] * u + (a[1] * v + a[2])
    ry_c = a[3] * u + (a[4] * v + a[5])
    rz_c = a[6] * u + (a[7] * v + a[8])
    for b in range(B):
        X = xp_ref[b, 0]
        Y = xp_ref[b, 1]
        Z = xp_ref[b, 2]
        M = xp_ref[b, 3]
        xc = X[1:1 + s1r, 1:401]
        yc = Y[1:1 + s1r, 1:401]
        zc = Z[1:1 + s1r, 1:401]
        mc = M[1:1 + s1r, 1:401]
        pairs = []
        for k in range(9):
            dr, dc = k // 3 - 1, k % 3 - 1
            zn = Z[1 + dr:1 + dr + s1r, 1 + dc:401 + dc]
            rx = rx_c + sx[k]
            ry = ry_c + sy[k]
            rz = rz_c + sz[k]
            dx = rx * zn - xc
            dy = ry * zn - yc
            dz = rz * zn - zc
            d = dx * dx + dy * dy + dz * dz
            pairs.append((d, jnp.full(d.shape, k, jnp.int32)))
        while len(pairs) > 1:
            nxt = []
            for i in range(0, len(pairs) - 1, 2):
                (da, ka), (db, kb) = pairs[i], pairs[i + 1]
                m = db < da
                nxt.append((jnp.where(m, db, da), jnp.where(m, kb, ka)))
            if len(pairs) % 2:
                nxt.append(pairs[-1])
            pairs = nxt
        best_k = pairs[0][1]
        rmask = mc > OFF_THRESH
        zmask = zc > 0.0
        ixz = jnp.where(rmask, jnp.where(zmask, best_k, 4), 9)
        irm = jnp.where(rmask & zmask, best_k, 4)

        def s1s(P, s, t):
            return P[2 - s:2 - s + R_TC, 2 - t:2 - t + W]

        def pls(P, s, t):
            return P[3 - s:3 - s + R_TC, 3 - t:3 - t + W]

        ixz_c = ixz[1:1 + R_TC, 1:1 + W]
        irm_c = irm[1:1 + R_TC, 1:1 + W]
        inf2 = jnp.full((R_TC, W), INF, jnp.float32)
        zero2 = jnp.zeros((R_TC, W), jnp.float32)
        c4 = ixz_c == 4
        bx = jnp.where(c4, pls(X, 1, 1), zero2)
        by = jnp.where(c4, pls(Y, 1, 1), zero2)
        bz = jnp.where(c4, pls(Z, 1, 1), zero2)
        brm = jnp.where(irm_c == 4, pls(M, 1, 1), zero2)
        best = inf2
        for k in range(9):
            s, t = k // 3, k % 3
            ixz_q = s1s(ixz, s, t)
            zq = pls(Z, s, t)
            cand = jnp.where(ixz_q == k, zq, zero2)
            cand = jnp.where(cand > 0.0, cand, inf2)
            m = cand < best
            best = jnp.where(m, cand, best)
            bx = jnp.where(m, pls(X, s, t), bx)
            by = jnp.where(m, pls(Y, s, t), by)
            bz = jnp.where(m, zq, bz)
            brm = jnp.where(m, pls(M, s, t), brm)
        out_ref[b, 0] = bx
        out_ref[b, 1] = by
        out_ref[b, 2] = bz
        out_ref[b, 3] = brm


_smap_tc = pl.pallas_call(
    _tc_body,
    out_shape=jax.ShapeDtypeStruct((B, 4, R_TC, W), jnp.float32),
    in_specs=[
        pl.BlockSpec(memory_space=pltpu.VMEM),
        pl.BlockSpec(memory_space=pltpu.SMEM),
    ],
    out_specs=pl.BlockSpec(memory_space=pltpu.VMEM),
    name="smap_tc",
)


def kernel(x, camera_matrix):
    k_inv = jnp.linalg.inv(camera_matrix)
    dc = jnp.array([k % 3 - 1 for k in range(9)], jnp.float32)
    dr = jnp.array([k // 3 - 1 for k in range(9)], jnp.float32)
    sx = k_inv[0, 0] * dc + k_inv[0, 1] * dr
    sy = k_inv[1, 0] * dc + k_inv[1, 1] * dr
    sz = k_inv[2, 0] * dc + k_inv[2, 1] * dr
    consts = jnp.concatenate([k_inv.reshape(9), sx, sy, sz])
    xp = jnp.pad(x, ((0, 0), (0, 0), (2, 2), (2, WBUF - W - 2)))
    xp2 = xp.reshape(B * 4 * (HP + 2), WBUF)
    consts16 = jnp.repeat(consts.reshape(36, 1), 16, axis=1)
    out_sc = _smap_sc(xp2, consts16)
    out_tc = _smap_tc(xp, consts)
    return lax.dynamic_update_slice(
        out_sc.reshape(B, 4, H, W), out_tc, (0, 0, 0, 0))
